# baseline (speedup 1.0000x reference)
import functools

import jax
import jax.numpy as jnp
from jax import lax
from jax.experimental import pallas as pl
from jax.experimental.pallas import tpu as pltpu
from jax.experimental.pallas import tpu_sc as plsc

_B = 16384
_D = 64
_CHUNK = 64
_G = 16


@functools.cache
def _make_sc_kernel():
  info = plsc.get_sparse_core_info()
  nc, ns = info.num_cores, info.num_subcores
  nw = nc * ns
  bpw = _B // nw
  nchunk = bpw // _CHUNK

  mesh = plsc.VectorSubcoreMesh(core_axis_name="c", subcore_axis_name="s")

  @functools.partial(
      pl.kernel,
      mesh=mesh,
      compiler_params=pltpu.CompilerParams(needs_layout_passes=False),
      out_type=jax.ShapeDtypeStruct((_B,), jnp.float32),
      scratch_types=[
          pltpu.VMEM((bpw,), jnp.int32),
          pltpu.VMEM((bpw,), jnp.int32),
          pltpu.VMEM((6 * _CHUNK // 8, 8, _D), jnp.float32),
          pltpu.VMEM((6 * _CHUNK // 8, 8, _D), jnp.float32),
          pltpu.VMEM((bpw,), jnp.float32),
          pltpu.VMEM((bpw,), jnp.float32),
          pltpu.VMEM((bpw,), jnp.float32),
          pltpu.SemaphoreType.DMA,
          pltpu.SemaphoreType.DMA,
          pltpu.SemaphoreType.DMA,
          pltpu.SemaphoreType.DMA,
          pltpu.SemaphoreType.DMA,
          pltpu.SemaphoreType.DMA,
          pltpu.SemaphoreType.DMA,
          pltpu.SemaphoreType.DMA,
      ],
  )
  def k(lidx_hbm, ridx_hbm, tab_hbm, bias_lhs_hbm, bias_rhs_hbm,
        out_hbm, lidx_v, ridx_v, lbuf_v, rbuf_v, lb_v, rb_v,
        out_v, sem0, sem1, sem2, sem3, sem4, sem5, sem6, sem7):
    sems = [sem0, sem1, sem2, sem3, sem4, sem5, sem6, sem7]
    wid = lax.axis_index("s") * nc + lax.axis_index("c")
    base = pl.multiple_of(wid * bpw, 8)
    pltpu.sync_copy(lidx_hbm.at[pl.ds(base, bpw)], lidx_v)
    pltpu.sync_copy(ridx_hbm.at[pl.ds(base, bpw)], ridx_v)

    copies = {}

    def fire(j):
      slot = j % 6

      def issue(g, carry, j=j, slot=slot):
        ivl = lidx_v[pl.ds(j * _CHUNK + g * _G, _G)]
        ivr = ridx_v[pl.ds(j * _CHUNK + g * _G, _G)]
        for u in range(_G):
          row = slot * _CHUNK + g * _G + u
          il, ir = ivl[u], ivr[u]
          pltpu.async_copy(tab_hbm.at[il >> 3, il & 7],
                           lbuf_v.at[row >> 3, row & 7], sems[j])
          pltpu.async_copy(tab_hbm.at[ir >> 3, ir & 7],
                           rbuf_v.at[row >> 3, row & 7], sems[j])
        return carry

      lax.fori_loop(0, _CHUNK // _G, issue, 0)
      copies[j] = [
          pltpu.async_copy(
              bias_lhs_hbm.at[lidx_v.at[pl.ds(j * _CHUNK, _CHUNK)]],
              lb_v.at[pl.ds(j * _CHUNK, _CHUNK)], sems[j]),
          pltpu.async_copy(
              bias_rhs_hbm.at[ridx_v.at[pl.ds(j * _CHUNK, _CHUNK)]],
              rb_v.at[pl.ds(j * _CHUNK, _CHUNK)], sems[j]),
      ]

    def drain(j):
      slot = j % 6
      for c in copies[j]:
        c.wait()
      pltpu.make_async_copy(
          tab_hbm.at[pl.ds(0, _CHUNK // 8)],
          lbuf_v.at[pl.ds(slot * (_CHUNK // 8), _CHUNK // 8)],
          sems[j]).wait()
      pltpu.make_async_copy(
          tab_hbm.at[pl.ds(0, _CHUNK // 8)],
          rbuf_v.at[pl.ds(slot * (_CHUNK // 8), _CHUNK // 8)],
          sems[j]).wait()

    for j in range(6):
      fire(j)
    for j in range(nchunk):
      drain(j)
      slot = j % 6

      def group(g, carry, j=j, slot=slot):
        lane = lax.iota(jnp.int32, _G)
        sq_vec = jnp.zeros((_G,), jnp.float32)
        for u in range(_G):
          p = g * _G + u
          row = slot * _CHUNK + p
          acc = jnp.zeros((_G,), jnp.float32)
          for c in range(_D // _G):
            lv = lbuf_v[row >> 3, row & 7, pl.ds(c * _G, _G)]
            rv = rbuf_v[row >> 3, row & 7, pl.ds(c * _G, _G)]
            d = lv - rv
            acc = acc + d * d
          s = jnp.sum(acc)
          sq_vec = jnp.where(lane == u, jnp.full((_G,), s, jnp.float32),
                             sq_vec)
        lb = lb_v[pl.ds(j * _CHUNK + g * _G, _G)]
        rb = rb_v[pl.ds(j * _CHUNK + g * _G, _G)]
        out_v[pl.ds(j * _CHUNK + g * _G, _G)] = (lb + rb) - (sq_vec + 1e-12)
        return carry

      lax.fori_loop(0, _CHUNK // _G, group, 0)
      if j + 6 < nchunk:
        fire(j + 6)

    pltpu.sync_copy(out_v, out_hbm.at[pl.ds(base, bpw)])

  return k


def kernel(input_triplet, table, bias_lhs, bias_rhs):
  k = _make_sc_kernel()
  tab3 = table.reshape(table.shape[0] // 8, 8, table.shape[1])
  lhs = input_triplet[:, 0].astype(jnp.int32)
  rhs = input_triplet[:, -1].astype(jnp.int32)
  return k(lhs, rhs, tab3, bias_lhs, bias_rhs)


# speedup vs baseline: 1.0266x; 1.0266x over previous
import functools

import jax
import jax.numpy as jnp
from jax import lax
from jax.experimental import pallas as pl
from jax.experimental.pallas import tpu as pltpu
from jax.experimental.pallas import tpu_sc as plsc

_B = 16384
_D = 64
_CHUNK = 128
_G = 16


@functools.cache
def _make_sc_kernel():
  info = plsc.get_sparse_core_info()
  nc, ns = info.num_cores, info.num_subcores
  nw = nc * ns
  bpw = _B // nw
  nchunk = bpw // _CHUNK

  mesh = plsc.VectorSubcoreMesh(core_axis_name="c", subcore_axis_name="s")

  @functools.partial(
      pl.kernel,
      mesh=mesh,
      compiler_params=pltpu.CompilerParams(needs_layout_passes=False),
      out_type=jax.ShapeDtypeStruct((_B,), jnp.float32),
      scratch_types=[
          pltpu.VMEM((bpw,), jnp.int32),
          pltpu.VMEM((bpw,), jnp.int32),
          pltpu.VMEM((3 * _CHUNK // 8, 8, _D), jnp.float32),
          pltpu.VMEM((3 * _CHUNK // 8, 8, _D), jnp.float32),
          pltpu.VMEM((bpw,), jnp.float32),
          pltpu.VMEM((bpw,), jnp.float32),
          pltpu.VMEM((bpw,), jnp.float32),
          pltpu.SemaphoreType.DMA,
          pltpu.SemaphoreType.DMA,
          pltpu.SemaphoreType.DMA,
          pltpu.SemaphoreType.DMA,
      ],
  )
  def k(lidx_hbm, ridx_hbm, tab_hbm, bias_lhs_hbm, bias_rhs_hbm,
        out_hbm, lidx_v, ridx_v, lbuf_v, rbuf_v, lb_v, rb_v,
        out_v, sem0, sem1, sem2, sem3):
    sems = [sem0, sem1, sem2, sem3]
    wid = lax.axis_index("s") * nc + lax.axis_index("c")
    base = pl.multiple_of(wid * bpw, 8)
    pltpu.sync_copy(lidx_hbm.at[pl.ds(base, bpw)], lidx_v)
    pltpu.sync_copy(ridx_hbm.at[pl.ds(base, bpw)], ridx_v)

    copies = {}

    def fire(j):
      slot = j % 3

      def issue(g, carry, j=j, slot=slot):
        ivl = lidx_v[pl.ds(j * _CHUNK + g * _G, _G)]
        ivr = ridx_v[pl.ds(j * _CHUNK + g * _G, _G)]
        for u in range(_G):
          row = slot * _CHUNK + g * _G + u
          il, ir = ivl[u], ivr[u]
          pltpu.async_copy(tab_hbm.at[il >> 3, il & 7],
                           lbuf_v.at[row >> 3, row & 7], sems[j])
          pltpu.async_copy(tab_hbm.at[ir >> 3, ir & 7],
                           rbuf_v.at[row >> 3, row & 7], sems[j])
        return carry

      lax.fori_loop(0, _CHUNK // _G, issue, 0)
      copies[j] = [
          pltpu.async_copy(
              bias_lhs_hbm.at[lidx_v.at[pl.ds(j * _CHUNK, _CHUNK)]],
              lb_v.at[pl.ds(j * _CHUNK, _CHUNK)], sems[j]),
          pltpu.async_copy(
              bias_rhs_hbm.at[ridx_v.at[pl.ds(j * _CHUNK, _CHUNK)]],
              rb_v.at[pl.ds(j * _CHUNK, _CHUNK)], sems[j]),
      ]

    def drain(j):
      slot = j % 3
      for c in copies[j]:
        c.wait()
      pltpu.make_async_copy(
          tab_hbm.at[pl.ds(0, _CHUNK // 8)],
          lbuf_v.at[pl.ds(slot * (_CHUNK // 8), _CHUNK // 8)],
          sems[j]).wait()
      pltpu.make_async_copy(
          tab_hbm.at[pl.ds(0, _CHUNK // 8)],
          rbuf_v.at[pl.ds(slot * (_CHUNK // 8), _CHUNK // 8)],
          sems[j]).wait()

    for j in range(3):
      fire(j)
    for j in range(nchunk):
      drain(j)
      slot = j % 3

      def group(g, carry, j=j, slot=slot):
        lane = lax.iota(jnp.int32, _G)
        sq_vec = jnp.zeros((_G,), jnp.float32)
        for u in range(_G):
          p = g * _G + u
          row = slot * _CHUNK + p
          acc = jnp.zeros((_G,), jnp.float32)
          for c in range(_D // _G):
            lv = lbuf_v[row >> 3, row & 7, pl.ds(c * _G, _G)]
            rv = rbuf_v[row >> 3, row & 7, pl.ds(c * _G, _G)]
            d = lv - rv
            acc = acc + d * d
          s = jnp.sum(acc)
          sq_vec = jnp.where(lane == u, jnp.full((_G,), s, jnp.float32),
                             sq_vec)
        lb = lb_v[pl.ds(j * _CHUNK + g * _G, _G)]
        rb = rb_v[pl.ds(j * _CHUNK + g * _G, _G)]
        out_v[pl.ds(j * _CHUNK + g * _G, _G)] = (lb + rb) - (sq_vec + 1e-12)
        return carry

      lax.fori_loop(0, _CHUNK // _G, group, 0)
      if j + 3 < nchunk:
        fire(j + 3)

    pltpu.sync_copy(out_v, out_hbm.at[pl.ds(base, bpw)])

  return k


def kernel(input_triplet, table, bias_lhs, bias_rhs):
  k = _make_sc_kernel()
  tab3 = table.reshape(table.shape[0] // 8, 8, table.shape[1])
  lhs = input_triplet[:, 0].astype(jnp.int32)
  rhs = input_triplet[:, -1].astype(jnp.int32)
  return k(lhs, rhs, tab3, bias_lhs, bias_rhs)
